# baseline (speedup 1.0000x reference)
import jax
import jax.numpy as jnp
from jax.experimental import pallas as pl
from jax.experimental.pallas import tpu as pltpu

B = 16384
D = 32
H = 64
N = 1_000_000
LANES = 128
IPB = 8


def _body(uidx_ref, iidx_ref, *refs):
    t_refs = refs[: 2 * IPB]
    w1u_ref, w1i_ref, b1_ref, w2_ref, b2_ref, o_ref = refs[2 * IPB:]
    g = pl.program_id(0)
    lane_iota = jax.lax.broadcasted_iota(jnp.int32, (1, LANES), 1)

    def extract(t_ref, idx):
        lane = idx % LANES
        base = idx - lane
        blk = t_ref[...]
        blk = jnp.where(base + lane_iota < N, blk, 0.0)
        e = (lane_iota == lane).astype(jnp.float32)
        return jax.lax.dot_general(e, blk, (((1,), (1,)), ((), ())),
                                   preferred_element_type=jnp.float32)

    urows = [extract(t_refs[k], uidx_ref[g * IPB + k]) for k in range(IPB)]
    irows = [extract(t_refs[IPB + k], iidx_ref[g * IPB + k])
             for k in range(IPB)]
    u = jnp.concatenate(urows, axis=0)
    i = jnp.concatenate(irows, axis=0)
    h = jnp.dot(u, w1u_ref[...], preferred_element_type=jnp.float32)
    h = h + jnp.dot(i, w1i_ref[...], preferred_element_type=jnp.float32)
    h = jnp.maximum(h + b1_ref[...], 0.0)
    o_ref[...] = jnp.dot(h, w2_ref[...],
                         preferred_element_type=jnp.float32) + b2_ref[...]


def kernel(user, item, user_table, item_table, W1, b1, W2, b2):
    uT = user_table.T
    iT = item_table.T
    user = user.astype(jnp.int32)
    item = item.astype(jnp.int32)

    def tile_spec(scalar_slot, k):
        def index_map(g, u_idx, i_idx):
            idx = (u_idx if scalar_slot == 0 else i_idx)[g * IPB + k]
            return (0, idx // LANES)
        return pl.BlockSpec((D, LANES), index_map)

    in_specs = (
        [tile_spec(0, k) for k in range(IPB)]
        + [tile_spec(1, k) for k in range(IPB)]
        + [
            pl.BlockSpec((D, H), lambda g, u, i: (0, 0)),
            pl.BlockSpec((D, H), lambda g, u, i: (0, 0)),
            pl.BlockSpec((1, H), lambda g, u, i: (0, 0)),
            pl.BlockSpec((H, 1), lambda g, u, i: (0, 0)),
            pl.BlockSpec((1, 1), lambda g, u, i: (0, 0)),
        ]
    )
    grid_spec = pltpu.PrefetchScalarGridSpec(
        num_scalar_prefetch=2,
        grid=(B // IPB,),
        in_specs=in_specs,
        out_specs=pl.BlockSpec((IPB, 1), lambda g, u, i: (g, 0)),
    )
    out = pl.pallas_call(
        _body,
        grid_spec=grid_spec,
        out_shape=jax.ShapeDtypeStruct((B, 1), jnp.float32),
    )(user, item,
      *([uT] * IPB), *([iT] * IPB),
      W1[:D], W1[D:], b1.reshape(1, H), W2, b2.reshape(1, 1))
    return out[:, 0]


# speedup vs baseline: 1.1849x; 1.1849x over previous
import jax
import jax.numpy as jnp
from jax.experimental import pallas as pl
from jax.experimental.pallas import tpu as pltpu

B = 16384
D = 32
H = 64
N = 1_000_000
LANES = 128
IPB = 32


def _body(uidx_ref, iidx_ref, *refs):
    t_refs = refs[: 2 * IPB]
    w1u_ref, w1i_ref, b1_ref, w2_ref, b2_ref, o_ref = refs[2 * IPB:]
    g = pl.program_id(0)
    lane_iota = jax.lax.broadcasted_iota(jnp.int32, (1, LANES), 1)

    def extract(t_ref, idx):
        lane = idx % LANES
        base = idx - lane
        blk = t_ref[...]
        blk = jnp.where(base + lane_iota < N, blk, 0.0)
        e = (lane_iota == lane).astype(jnp.float32)
        return jax.lax.dot_general(e, blk, (((1,), (1,)), ((), ())),
                                   preferred_element_type=jnp.float32)

    urows = [extract(t_refs[k], uidx_ref[g * IPB + k]) for k in range(IPB)]
    irows = [extract(t_refs[IPB + k], iidx_ref[g * IPB + k])
             for k in range(IPB)]
    u = jnp.concatenate(urows, axis=0)
    i = jnp.concatenate(irows, axis=0)
    h = jnp.dot(u, w1u_ref[...], preferred_element_type=jnp.float32)
    h = h + jnp.dot(i, w1i_ref[...], preferred_element_type=jnp.float32)
    h = jnp.maximum(h + b1_ref[...], 0.0)
    o_ref[...] = jnp.dot(h, w2_ref[...],
                         preferred_element_type=jnp.float32) + b2_ref[...]


def kernel(user, item, user_table, item_table, W1, b1, W2, b2):
    uT = user_table.T
    iT = item_table.T
    user = user.astype(jnp.int32)
    item = item.astype(jnp.int32)

    def tile_spec(scalar_slot, k):
        def index_map(g, u_idx, i_idx):
            idx = (u_idx if scalar_slot == 0 else i_idx)[g * IPB + k]
            return (0, idx // LANES)
        return pl.BlockSpec((D, LANES), index_map)

    in_specs = (
        [tile_spec(0, k) for k in range(IPB)]
        + [tile_spec(1, k) for k in range(IPB)]
        + [
            pl.BlockSpec((D, H), lambda g, u, i: (0, 0)),
            pl.BlockSpec((D, H), lambda g, u, i: (0, 0)),
            pl.BlockSpec((1, H), lambda g, u, i: (0, 0)),
            pl.BlockSpec((H, 1), lambda g, u, i: (0, 0)),
            pl.BlockSpec((1, 1), lambda g, u, i: (0, 0)),
        ]
    )
    grid_spec = pltpu.PrefetchScalarGridSpec(
        num_scalar_prefetch=2,
        grid=(B // IPB,),
        in_specs=in_specs,
        out_specs=pl.BlockSpec((IPB, 1), lambda g, u, i: (g, 0)),
    )
    out = pl.pallas_call(
        _body,
        grid_spec=grid_spec,
        out_shape=jax.ShapeDtypeStruct((B, 1), jnp.float32),
    )(user, item,
      *([uT] * IPB), *([iT] * IPB),
      W1[:D], W1[D:], b1.reshape(1, H), W2, b2.reshape(1, 1))
    return out[:, 0]
